# baseline (speedup 1.0000x reference)
import jax
import jax.numpy as jnp
from jax.experimental import pallas as pl

N = 50000
HOPS = 3


def kernel(user_emb, item_emb, adj_indices, adj_values,
           W_gc_0, b_gc_0, W_bi_0, b_bi_0,
           W_gc_1, b_gc_1, W_bi_1, b_bi_1,
           W_gc_2, b_gc_2, W_bi_2, b_bi_2):
    Ws_gc = [W_gc_0, W_gc_1, W_gc_2]
    bs_gc = [b_gc_0, b_gc_1, b_gc_2]
    Ws_bi = [W_bi_0, W_bi_1, W_bi_2]
    bs_bi = [b_bi_0, b_bi_1, b_bi_2]
    row = adj_indices[0]
    col = adj_indices[1]
    ego = jnp.concatenate([user_emb, item_emb], axis=0)
    all_embs = [ego]
    for k in range(HOPS):
        gathered = adj_values[:, None] * jnp.take(ego, col, axis=0)
        side = jax.ops.segment_sum(gathered, row, num_segments=N)
        sum_emb = jax.nn.leaky_relu(side @ Ws_gc[k] + bs_gc[k], negative_slope=0.2)
        bi_emb = jax.nn.leaky_relu((ego * side) @ Ws_bi[k] + bs_bi[k], negative_slope=0.2)
        ego = sum_emb + bi_emb
        norm = ego / (jnp.linalg.norm(ego, axis=1, keepdims=True) + 1e-12)
        all_embs.append(norm)
    return jnp.stack(all_embs, axis=1)


# speedup vs baseline: 6.6615x; 6.6615x over previous
import functools

import jax
import jax.numpy as jnp
from jax import lax
from jax.experimental import pallas as pl
from jax.experimental.pallas import tpu as pltpu
from jax.experimental.pallas import tpu_sc as plsc

N = 50000
E = 800000
D = 64
DH = 32
HOPS = 3

NUM_CORES = 2
NUM_TILES = 16
CH = 80
KR = 16
SUPER = 40
CROWS_PER_TILE = KR * SUPER
E_PAD = NUM_TILES * CROWS_PER_TILE * CH
N_PAD = 50048
ROWS_PER_TILE = N_PAD // NUM_TILES
ZROWS = 136

_mesh = plsc.VectorSubcoreMesh(core_axis_name="c", subcore_axis_name="s")


@functools.partial(
    pl.kernel,
    out_type=jax.ShapeDtypeStruct((NUM_CORES, N_PAD, DH), jnp.float32),
    mesh=_mesh,
    compiler_params=pltpu.CompilerParams(use_tc_tiling_on_sc=False),
    scratch_types=[
        pltpu.VMEM_SHARED((N_PAD, DH), jnp.float32),
        pltpu.VMEM((KR, CH), jnp.int32),
        pltpu.VMEM((KR, CH), jnp.int32),
        pltpu.VMEM((KR * CH,), jnp.float32),
        pltpu.VMEM((CH, DH), jnp.float32),
        pltpu.VMEM((CH, DH), jnp.float32),
        pltpu.VMEM((ZROWS, DH), jnp.float32),
        pltpu.SemaphoreType.DMA,
        pltpu.SemaphoreType.DMA,
    ],
)
def _spmv_sc(ego2, colm, rowm, valm, out, acc, colv, rowv, valv,
             gbuf0, gbuf1, zbuf, sem0, sem1):
    c = lax.axis_index("c")
    s = lax.axis_index("s")

    zeros16 = jnp.zeros((16,), jnp.float32)

    @plsc.parallel_loop(0, ZROWS)
    def _(j):
        zbuf[j, pl.ds(0, 16)] = zeros16
        zbuf[j, pl.ds(16, 16)] = zeros16

    row0 = s * ROWS_PER_TILE

    def zero_body(i, _):
        pltpu.sync_copy(zbuf, acc.at[pl.ds(row0 + i * ZROWS, ZROWS)])
        return 0

    lax.fori_loop(0, ROWS_PER_TILE // ZROWS, zero_body, 0)

    plsc.subcore_barrier()

    gbufs = (gbuf0, gbuf1)
    sems = (sem0, sem1)
    tile_crow = s * CROWS_PER_TILE

    def super_body(sc_i, _):
        crow = tile_crow + sc_i * KR
        pltpu.sync_copy(colm.at[pl.ds(crow, KR)], colv)
        pltpu.sync_copy(rowm.at[pl.ds(crow, KR)], rowv)
        pltpu.sync_copy(valm.at[pl.ds(crow * CH, KR * CH)], valv)

        @plsc.parallel_loop(0, KR)
        def _(r):
            for q in range(CH // 16):
                v = colv[r, pl.ds(q * 16, 16)]
                colv[r, pl.ds(q * 16, 16)] = v * 2 + c

        pltpu.async_copy(ego2.at[colv.at[0]], gbufs[0], sems[0])
        for k in range(KR):
            kb = k & 1
            if k + 1 < KR:
                pltpu.async_copy(ego2.at[colv.at[k + 1]], gbufs[1 - kb],
                                 sems[1 - kb])
            pltpu.make_async_copy(ego2.at[colv.at[k]], gbufs[kb],
                                  sems[kb]).wait()
            g = gbufs[kb]

            @plsc.parallel_loop(0, CH // 16)
            def _(q):
                vv = valv[pl.ds(k * CH + q * 16, 16)]
                for j in range(16):
                    v = jnp.full((16,), vv[j])
                    e = q * 16 + j
                    g[e, pl.ds(0, 16)] = g[e, pl.ds(0, 16)] * v
                    g[e, pl.ds(16, 16)] = g[e, pl.ds(16, 16)] * v

            pltpu.sync_copy(g, acc.at[rowv.at[k]], add=True)
        return 0

    lax.fori_loop(0, SUPER, super_body, 0)

    plsc.subcore_barrier()

    pltpu.sync_copy(acc.at[pl.ds(row0, ROWS_PER_TILE)],
                    out.at[c, pl.ds(row0, ROWS_PER_TILE)])


BLK = 2000


def _tc_body(h0, h1, ego, wg, bg, wb, bb, eout, nout):
    side = jnp.concatenate([h0[0], h1[0]], axis=1)
    e = ego[...]
    sm = jnp.dot(side, wg[...], preferred_element_type=jnp.float32) + bg[...]
    sm = jnp.where(sm >= 0, sm, 0.2 * sm)
    bm = jnp.dot(e * side, wb[...], preferred_element_type=jnp.float32) + bb[...]
    bm = jnp.where(bm >= 0, bm, 0.2 * bm)
    o = sm + bm
    eout[...] = o
    ss = jnp.sum(o * o, axis=1, keepdims=True)
    nout[...] = o / (jnp.sqrt(ss) + 1e-12)


_tc_dense = pl.pallas_call(
    _tc_body,
    grid=(N // BLK,),
    in_specs=[
        pl.BlockSpec((1, BLK, DH), lambda i: (0, i, 0)),
        pl.BlockSpec((1, BLK, DH), lambda i: (1, i, 0)),
        pl.BlockSpec((BLK, D), lambda i: (i, 0)),
        pl.BlockSpec((D, D), lambda i: (0, 0)),
        pl.BlockSpec((1, D), lambda i: (0, 0)),
        pl.BlockSpec((D, D), lambda i: (0, 0)),
        pl.BlockSpec((1, D), lambda i: (0, 0)),
    ],
    out_specs=[
        pl.BlockSpec((BLK, D), lambda i: (i, 0)),
        pl.BlockSpec((BLK, D), lambda i: (i, 0)),
    ],
    out_shape=[
        jax.ShapeDtypeStruct((N, D), jnp.float32),
        jax.ShapeDtypeStruct((N, D), jnp.float32),
    ],
)


def kernel(user_emb, item_emb, adj_indices, adj_values,
           W_gc_0, b_gc_0, W_bi_0, b_bi_0,
           W_gc_1, b_gc_1, W_bi_1, b_bi_1,
           W_gc_2, b_gc_2, W_bi_2, b_bi_2):
    Ws_gc = [W_gc_0, W_gc_1, W_gc_2]
    bs_gc = [b_gc_0, b_gc_1, b_gc_2]
    Ws_bi = [W_bi_0, W_bi_1, W_bi_2]
    bs_bi = [b_bi_0, b_bi_1, b_bi_2]

    pad = E_PAD - E
    spread = jnp.arange(pad, dtype=jnp.int32) % N
    col_p = jnp.concatenate([adj_indices[1].astype(jnp.int32), spread])
    row_p = jnp.concatenate([adj_indices[0].astype(jnp.int32), spread])
    val_p = jnp.concatenate([adj_values, jnp.zeros((pad,), jnp.float32)])
    colm = col_p.reshape(E_PAD // CH, CH)
    rowm = row_p.reshape(E_PAD // CH, CH)

    ego = jnp.concatenate([user_emb, item_emb], axis=0)
    all_embs = [ego]
    for k in range(HOPS):
        ego2 = ego.reshape(2 * N, DH)
        side2 = _spmv_sc(ego2, colm, rowm, val_p)
        ego, norm = _tc_dense(side2, side2, ego,
                              Ws_gc[k], bs_gc[k], Ws_bi[k], bs_bi[k])
        all_embs.append(norm)
    return jnp.stack(all_embs, axis=1)


# speedup vs baseline: 8.8024x; 1.3214x over previous
import functools

import jax
import jax.numpy as jnp
from jax import lax
from jax.experimental import pallas as pl
from jax.experimental.pallas import tpu as pltpu
from jax.experimental.pallas import tpu_sc as plsc

N = 50000
E = 800000
D = 64
DH = 32
HOPS = 3

NUM_CORES = 2
NUM_TILES = 16
CH = 128
KR = 16
SUPER = 25
NB = 4
CROWS_PER_TILE = KR * SUPER
E_PAD = NUM_TILES * CROWS_PER_TILE * CH
N_PAD = 50048
ROWS_PER_TILE = N_PAD // NUM_TILES
ZROWS = 136

_mesh = plsc.VectorSubcoreMesh(core_axis_name="c", subcore_axis_name="s")


@functools.partial(
    pl.kernel,
    out_type=jax.ShapeDtypeStruct((NUM_CORES, N_PAD, DH), jnp.float32),
    mesh=_mesh,
    compiler_params=pltpu.CompilerParams(use_tc_tiling_on_sc=False),
    scratch_types=[
        pltpu.VMEM_SHARED((N_PAD, DH), jnp.float32),
        pltpu.VMEM((KR, CH), jnp.int32),
        pltpu.VMEM((KR, CH), jnp.int32),
        pltpu.VMEM((KR * CH,), jnp.float32),
        [pltpu.VMEM((CH, DH), jnp.float32) for _ in range(NB)],
        pltpu.VMEM((ZROWS, DH), jnp.float32),
        [pltpu.SemaphoreType.DMA for _ in range(NB)],
        [pltpu.SemaphoreType.DMA for _ in range(NB)],
    ],
)
def _spmv_sc(ego2, colm, rowm, valm, out, acc, colv, rowv, valv,
             gbufs, zbuf, gsems, ssems):
    c = lax.axis_index("c")
    s = lax.axis_index("s")

    zeros16 = jnp.zeros((16,), jnp.float32)

    @plsc.parallel_loop(0, ZROWS)
    def _(j):
        zbuf[j, pl.ds(0, 16)] = zeros16
        zbuf[j, pl.ds(16, 16)] = zeros16

    row0 = s * ROWS_PER_TILE

    def zero_body(i, _):
        pltpu.sync_copy(zbuf, acc.at[pl.ds(row0 + i * ZROWS, ZROWS)])
        return 0

    lax.fori_loop(0, ROWS_PER_TILE // ZROWS, zero_body, 0)

    plsc.subcore_barrier()

    tile_crow = s * CROWS_PER_TILE

    def super_body(sc_i, _):
        crow = tile_crow + sc_i * KR
        pltpu.sync_copy(colm.at[pl.ds(crow, KR)], colv)
        pltpu.sync_copy(rowm.at[pl.ds(crow, KR)], rowv)
        pltpu.sync_copy(valm.at[pl.ds(crow * CH, KR * CH)], valv)

        @plsc.parallel_loop(0, KR)
        def _(r):
            for q in range(CH // 16):
                v = colv[r, pl.ds(q * 16, 16)]
                colv[r, pl.ds(q * 16, 16)] = v * 2 + c

        gd = [None] * KR
        sd = [None] * KR
        gd[0] = pltpu.async_copy(ego2.at[colv.at[0]], gbufs[0], gsems[0])
        for k in range(KR):
            kb = k % NB
            if k + 1 < KR:
                nb = (k + 1) % NB
                if k + 1 >= NB:
                    sd[k + 1 - NB].wait()
                gd[k + 1] = pltpu.async_copy(ego2.at[colv.at[k + 1]],
                                             gbufs[nb], gsems[nb])
            gd[k].wait()
            g = gbufs[kb]

            @plsc.parallel_loop(0, CH // 16)
            def _(q):
                vv = valv[pl.ds(k * CH + q * 16, 16)]
                for j in range(16):
                    v = jnp.full((16,), vv[j])
                    e = q * 16 + j
                    g[e, pl.ds(0, 16)] = g[e, pl.ds(0, 16)] * v
                    g[e, pl.ds(16, 16)] = g[e, pl.ds(16, 16)] * v

            sd[k] = pltpu.async_copy(g, acc.at[rowv.at[k]], ssems[kb],
                                     add=True)
        for k in range(KR - NB, KR):
            sd[k].wait()
        return 0

    lax.fori_loop(0, SUPER, super_body, 0)

    plsc.subcore_barrier()

    pltpu.sync_copy(acc.at[pl.ds(row0, ROWS_PER_TILE)],
                    out.at[c, pl.ds(row0, ROWS_PER_TILE)])


BLK = 2000


def _tc_body(h0, h1, ego, wg, bg, wb, bb, eout, nout):
    side = jnp.concatenate([h0[0], h1[0]], axis=1)
    e = ego[...]
    sm = jnp.dot(side, wg[...], preferred_element_type=jnp.float32) + bg[...]
    sm = jnp.where(sm >= 0, sm, 0.2 * sm)
    bm = jnp.dot(e * side, wb[...], preferred_element_type=jnp.float32) + bb[...]
    bm = jnp.where(bm >= 0, bm, 0.2 * bm)
    o = sm + bm
    eout[...] = o
    ss = jnp.sum(o * o, axis=1, keepdims=True)
    nout[...] = o / (jnp.sqrt(ss) + 1e-12)


_tc_dense = pl.pallas_call(
    _tc_body,
    grid=(N // BLK,),
    in_specs=[
        pl.BlockSpec((1, BLK, DH), lambda i: (0, i, 0)),
        pl.BlockSpec((1, BLK, DH), lambda i: (1, i, 0)),
        pl.BlockSpec((BLK, D), lambda i: (i, 0)),
        pl.BlockSpec((D, D), lambda i: (0, 0)),
        pl.BlockSpec((1, D), lambda i: (0, 0)),
        pl.BlockSpec((D, D), lambda i: (0, 0)),
        pl.BlockSpec((1, D), lambda i: (0, 0)),
    ],
    out_specs=[
        pl.BlockSpec((BLK, D), lambda i: (i, 0)),
        pl.BlockSpec((BLK, D), lambda i: (i, 0)),
    ],
    out_shape=[
        jax.ShapeDtypeStruct((N, D), jnp.float32),
        jax.ShapeDtypeStruct((N, D), jnp.float32),
    ],
)


def kernel(user_emb, item_emb, adj_indices, adj_values,
           W_gc_0, b_gc_0, W_bi_0, b_bi_0,
           W_gc_1, b_gc_1, W_bi_1, b_bi_1,
           W_gc_2, b_gc_2, W_bi_2, b_bi_2):
    Ws_gc = [W_gc_0, W_gc_1, W_gc_2]
    bs_gc = [b_gc_0, b_gc_1, b_gc_2]
    Ws_bi = [W_bi_0, W_bi_1, W_bi_2]
    bs_bi = [b_bi_0, b_bi_1, b_bi_2]

    pad = E_PAD - E
    spread = jnp.arange(pad, dtype=jnp.int32) % N
    col_p = jnp.concatenate([adj_indices[1].astype(jnp.int32), spread])
    row_p = jnp.concatenate([adj_indices[0].astype(jnp.int32), spread])
    val_p = jnp.concatenate([adj_values, jnp.zeros((pad,), jnp.float32)])
    colm = col_p.reshape(E_PAD // CH, CH)
    rowm = row_p.reshape(E_PAD // CH, CH)

    ego = jnp.concatenate([user_emb, item_emb], axis=0)
    all_embs = [ego]
    for k in range(HOPS):
        ego2 = ego.reshape(2 * N, DH)
        side2 = _spmv_sc(ego2, colm, rowm, val_p)
        ego, norm = _tc_dense(side2, side2, ego,
                              Ws_gc[k], bs_gc[k], Ws_bi[k], bs_bi[k])
        all_embs.append(norm)
    return jnp.stack(all_embs, axis=1)


# speedup vs baseline: 8.9648x; 1.0185x over previous
import functools

import jax
import jax.numpy as jnp
from jax import lax
from jax.experimental import pallas as pl
from jax.experimental.pallas import tpu as pltpu
from jax.experimental.pallas import tpu_sc as plsc

N = 50000
E = 800000
D = 64
DH = 32
HOPS = 3

NUM_CORES = 2
NUM_TILES = 16
CH = 128
KR = 8
SUPER = 52
NB = 4
CROWS_PER_TILE = KR * SUPER
CROWS = NUM_TILES * CROWS_PER_TILE
MAX_CROW = CROWS - KR
E_PAD = CROWS * CH
N_PAD = 50048
ROWS_PER_TILE = N_PAD // NUM_TILES
ZROWS = 92

_mesh = plsc.VectorSubcoreMesh(core_axis_name="c", subcore_axis_name="s")


@functools.partial(
    pl.kernel,
    out_type=jax.ShapeDtypeStruct((NUM_CORES, N_PAD, DH), jnp.float32),
    mesh=_mesh,
    compiler_params=pltpu.CompilerParams(use_tc_tiling_on_sc=False),
    scratch_types=[
        pltpu.VMEM_SHARED((N_PAD, DH), jnp.float32),
        [pltpu.VMEM((KR, CH), jnp.int32) for _ in range(2)],
        [pltpu.VMEM((KR, CH), jnp.int32) for _ in range(2)],
        [pltpu.VMEM((KR * CH,), jnp.float32) for _ in range(2)],
        [pltpu.VMEM((CH, DH), jnp.float32) for _ in range(NB)],
        pltpu.VMEM((ZROWS, DH), jnp.float32),
        [pltpu.SemaphoreType.DMA for _ in range(NB)],
        [pltpu.SemaphoreType.DMA for _ in range(NB)],
        [pltpu.SemaphoreType.DMA for _ in range(2)],
    ],
)
def _spmv_sc(ego2, colm2, rowm, valm, out, acc, colvs, rowvs, valvs,
             gbufs, zbuf, gsems, ssems, stsems):
    c = lax.axis_index("c")
    s = lax.axis_index("s")

    zeros16 = jnp.zeros((16,), jnp.float32)

    @plsc.parallel_loop(0, ZROWS)
    def _(j):
        zbuf[j, pl.ds(0, 16)] = zeros16
        zbuf[j, pl.ds(16, 16)] = zeros16

    row0 = s * ROWS_PER_TILE

    def zero_body(i, _):
        pltpu.sync_copy(zbuf, acc.at[pl.ds(row0 + i * ZROWS, ZROWS)])
        return 0

    lax.fori_loop(0, ROWS_PER_TILE // ZROWS, zero_body, 0)

    plsc.subcore_barrier()

    tile_crow = s * CROWS_PER_TILE

    def crow_of(m):
        return jnp.minimum(tile_crow + m * KR, MAX_CROW)

    def stage_descs(m, b):
        crow = crow_of(m)
        return (
            (colm2.at[c, pl.ds(crow, KR)], colvs[b], stsems[b]),
            (rowm.at[pl.ds(crow, KR)], rowvs[b], stsems[b]),
            (valm.at[pl.ds(crow * CH, KR * CH)], valvs[b], stsems[b]),
        )

    def stage_issue(m, b):
        for src, dst, sem in stage_descs(m, b):
            pltpu.async_copy(src, dst, sem)

    def stage_wait(m, b):
        for src, dst, sem in stage_descs(m, b):
            pltpu.make_async_copy(src, dst, sem).wait()

    def g0_issue(b):
        pltpu.async_copy(ego2.at[colvs[b].at[0]], gbufs[0], gsems[0])

    def g0_wait(b):
        pltpu.make_async_copy(ego2.at[colvs[b].at[0]], gbufs[0],
                              gsems[0]).wait()

    def process(i, a, b):
        colv, rowv, valv = colvs[a], rowvs[a], valvs[a]
        gd = [None] * (KR + 1)
        sd = [None] * KR
        for k in range(KR):
            kb = k % NB
            if k + 1 < KR:
                nb = (k + 1) % NB
                if k + 1 >= NB:
                    sd[k + 1 - NB].wait()
                gd[k + 1] = pltpu.async_copy(ego2.at[colv.at[k + 1]],
                                             gbufs[nb], gsems[nb])
            if k == 0:
                g0_wait(a)
            else:
                gd[k].wait()
            g = gbufs[kb]

            @plsc.parallel_loop(0, CH // 16, unroll=2)
            def _(q):
                vv = valv[pl.ds(k * CH + q * 16, 16)]
                for j in range(16):
                    v = jnp.full((16,), vv[j])
                    e = q * 16 + j
                    g[e, pl.ds(0, 16)] = g[e, pl.ds(0, 16)] * v
                    g[e, pl.ds(16, 16)] = g[e, pl.ds(16, 16)] * v

            sd[k] = pltpu.async_copy(g, acc.at[rowv.at[k]], ssems[kb],
                                     add=True)
        for k in range(KR - NB, KR):
            sd[k].wait()
        stage_wait(i + 1, b)
        g0_issue(b)
        stage_issue(i + 2, a)

    stage_issue(0, 0)
    stage_wait(0, 0)
    stage_issue(1, 1)
    g0_issue(0)

    def super_pair(j, _):
        process(2 * j, 0, 1)
        process(2 * j + 1, 1, 0)
        return 0

    lax.fori_loop(0, SUPER // 2, super_pair, 0)

    g0_wait(0)
    stage_wait(SUPER + 1, 1)

    plsc.subcore_barrier()

    pltpu.sync_copy(acc.at[pl.ds(row0, ROWS_PER_TILE)],
                    out.at[c, pl.ds(row0, ROWS_PER_TILE)])


BLK = 2000


def _tc_body(h0, h1, ego, wg, bg, wb, bb, eout, nout):
    side = jnp.concatenate([h0[0], h1[0]], axis=1)
    e = ego[...]
    sm = jnp.dot(side, wg[...], preferred_element_type=jnp.float32) + bg[...]
    sm = jnp.where(sm >= 0, sm, 0.2 * sm)
    bm = jnp.dot(e * side, wb[...], preferred_element_type=jnp.float32) + bb[...]
    bm = jnp.where(bm >= 0, bm, 0.2 * bm)
    o = sm + bm
    eout[...] = o
    ss = jnp.sum(o * o, axis=1, keepdims=True)
    nout[...] = o / (jnp.sqrt(ss) + 1e-12)


_tc_dense = pl.pallas_call(
    _tc_body,
    grid=(N // BLK,),
    in_specs=[
        pl.BlockSpec((1, BLK, DH), lambda i: (0, i, 0)),
        pl.BlockSpec((1, BLK, DH), lambda i: (1, i, 0)),
        pl.BlockSpec((BLK, D), lambda i: (i, 0)),
        pl.BlockSpec((D, D), lambda i: (0, 0)),
        pl.BlockSpec((1, D), lambda i: (0, 0)),
        pl.BlockSpec((D, D), lambda i: (0, 0)),
        pl.BlockSpec((1, D), lambda i: (0, 0)),
    ],
    out_specs=[
        pl.BlockSpec((BLK, D), lambda i: (i, 0)),
        pl.BlockSpec((BLK, D), lambda i: (i, 0)),
    ],
    out_shape=[
        jax.ShapeDtypeStruct((N, D), jnp.float32),
        jax.ShapeDtypeStruct((N, D), jnp.float32),
    ],
)


def kernel(user_emb, item_emb, adj_indices, adj_values,
           W_gc_0, b_gc_0, W_bi_0, b_bi_0,
           W_gc_1, b_gc_1, W_bi_1, b_bi_1,
           W_gc_2, b_gc_2, W_bi_2, b_bi_2):
    Ws_gc = [W_gc_0, W_gc_1, W_gc_2]
    bs_gc = [b_gc_0, b_gc_1, b_gc_2]
    Ws_bi = [W_bi_0, W_bi_1, W_bi_2]
    bs_bi = [b_bi_0, b_bi_1, b_bi_2]

    pad = E_PAD - E
    spread = jnp.arange(pad, dtype=jnp.int32) % N
    col_p = jnp.concatenate([adj_indices[1].astype(jnp.int32), spread])
    row_p = jnp.concatenate([adj_indices[0].astype(jnp.int32), spread])
    val_p = jnp.concatenate([adj_values, jnp.zeros((pad,), jnp.float32)])
    colm2 = (2 * col_p[None, :] +
             jnp.array([0, 1], jnp.int32)[:, None]).reshape(2, CROWS, CH)
    rowm = row_p.reshape(CROWS, CH)

    ego = jnp.concatenate([user_emb, item_emb], axis=0)
    all_embs = [ego]
    for k in range(HOPS):
        ego2 = ego.reshape(2 * N, DH)
        side2 = _spmv_sc(ego2, colm2, rowm, val_p)
        ego, norm = _tc_dense(side2, side2, ego,
                              Ws_gc[k], bs_gc[k], Ws_bi[k], bs_bi[k])
        all_embs.append(norm)
    return jnp.stack(all_embs, axis=1)
